# baseline (speedup 1.0000x reference)
import functools

import jax
import jax.numpy as jnp
from jax import lax
from jax.experimental import pallas as pl
from jax.experimental.pallas import tpu as pltpu
from jax.experimental.pallas import tpu_sc as plsc

VOCAB = 4
N_SEG = 2
B = 1024
S = 286
D = 284
NK = VOCAB * N_SEG
S2 = S // 2
D2 = 2 * D
NT2 = NK * NK * S2

NC = 2
NS = 16
NW = NC * NS

R2 = B * S2
SLABS = 4
R2S = R2 // SLABS
PER_W = R2S // NW
CHUNK = 104
NCHUNK = PER_W // CHUNK


def _ln(emb, gam, bet):
    mean = jnp.mean(emb, axis=-1, keepdims=True)
    cent = emb - mean
    var = jnp.mean(cent * cent, axis=-1, keepdims=True)
    return cent * lax.rsqrt(var + 1e-5) * gam + bet


def _rows_kernel(xe_ref, xo_ref, se_ref, so_ref, rows_ref):
    ke = xe_ref[...] * N_SEG + se_ref[...]
    ko = xo_ref[...] * N_SEG + so_ref[...]
    s_iota = lax.broadcasted_iota(jnp.int32, (B, S2), 1)
    rows_ref[...] = (ke * NK + ko) * S2 + s_iota


def _tc_rows(xe, xo, se, so):
    return pl.pallas_call(
        _rows_kernel,
        out_shape=jax.ShapeDtypeStruct((B, S2), jnp.int32),
    )(xe, xo, se, so)


def _table_kernel(tok_ref, pos_e_ref, pos_o_ref, seg_ref, gam_ref, bet_ref,
                  tab_ref):
    k = pl.program_id(0)
    k1, k2 = k // NK, k % NK
    v1, g1 = k1 // N_SEG, k1 % N_SEG
    v2, g2 = k2 // N_SEG, k2 % N_SEG
    gam = gam_ref[0][None, :]
    bet = bet_ref[0][None, :]
    comb_e = tok_ref[pl.ds(v1, 1), :] + seg_ref[pl.ds(g1, 1), :]
    comb_o = tok_ref[pl.ds(v2, 1), :] + seg_ref[pl.ds(g2, 1), :]
    out_e = _ln(comb_e + pos_e_ref[...], gam, bet)
    out_o = _ln(comb_o + pos_o_ref[...], gam, bet)
    tab_ref[0] = jnp.concatenate([out_e, out_o], axis=-1)


def _tc_table(tok, pos_e, pos_o, sege, gam, bet):
    full = lambda shape: pl.BlockSpec(shape, lambda k: (0,) * len(shape))
    return pl.pallas_call(
        _table_kernel,
        grid=(NK * NK,),
        in_specs=[
            full((VOCAB, D)),
            full((S2, D)),
            full((S2, D)),
            full((N_SEG, D)),
            full((1, D)),
            full((1, D)),
        ],
        out_specs=pl.BlockSpec((1, S2, D2), lambda k: (k, 0, 0)),
        out_shape=jax.ShapeDtypeStruct((NK * NK, S2, D2), jnp.float32),
    )(tok, pos_e, pos_o, sege, gam, bet)


@functools.cache
def _make_sc_gather():
    mesh = plsc.VectorSubcoreMesh(core_axis_name="c", subcore_axis_name="s")

    @functools.partial(
        pl.kernel,
        mesh=mesh,
        out_type=jax.ShapeDtypeStruct((R2S, D2), jnp.float32),
        scratch_types=[
            pltpu.VMEM((NCHUNK, CHUNK), jnp.int32),
            pltpu.VMEM((CHUNK, D2), jnp.float32),
            pltpu.VMEM((CHUNK, D2), jnp.float32),
            pltpu.SemaphoreType.DMA,
            pltpu.SemaphoreType.DMA,
        ],
        compiler_params=pltpu.CompilerParams(use_tc_tiling_on_sc=False),
    )
    def _sc_gather(tab_hbm, rows_hbm, out_hbm, idx_v, buf0, buf1, sem0, sem1):
        wid = lax.axis_index("s") * NC + lax.axis_index("c")
        base = wid * PER_W
        pltpu.sync_copy(rows_hbm.at[wid], idx_v)

        bufs = (buf0, buf1)
        sems = (sem0, sem1)

        def body(j2, _):
            copies = []
            for b in range(2):
                j = j2 * 2 + b
                copies.append(pltpu.async_copy(
                    tab_hbm.at[idx_v.at[j]], bufs[b], sems[b]))
            for b in range(2):
                j = j2 * 2 + b
                copies[b].wait()
                pltpu.sync_copy(bufs[b],
                                out_hbm.at[pl.ds(base + j * CHUNK, CHUNK)])
            return 0

        lax.fori_loop(0, NCHUNK // 2, body, 0)

        if NCHUNK % 2:
            j = NCHUNK - 1
            pltpu.async_copy(tab_hbm.at[idx_v.at[j]], buf0, sem0).wait()
            pltpu.sync_copy(buf0, out_hbm.at[pl.ds(base + j * CHUNK, CHUNK)])

    return _sc_gather


def kernel(x, seg, tok_embed, pos_embed, seg_embed, gamma, beta):
    rows2 = _tc_rows(x[:, 0::2], x[:, 1::2], seg[:, 0::2], seg[:, 1::2])
    tab = _tc_table(tok_embed, pos_embed[0::2], pos_embed[1::2], seg_embed,
                    gamma.reshape(1, D), beta.reshape(1, D))
    tab2 = tab.reshape(NT2, D2)
    rows4 = rows2.reshape(SLABS, NW, NCHUNK, CHUNK)
    gather = _make_sc_gather()
    bs = B // SLABS
    outs = [gather(tab2, rows4[i]).reshape(bs, S, D) for i in range(SLABS)]
    return jnp.concatenate(outs, axis=0)


# speedup vs baseline: 1.2922x; 1.2922x over previous
import functools

import jax
import jax.numpy as jnp
from jax import lax
from jax.experimental import pallas as pl
from jax.experimental.pallas import tpu as pltpu
from jax.experimental.pallas import tpu_sc as plsc

VOCAB = 4
N_SEG = 2
B = 1024
S = 286
D = 284
NK = VOCAB * N_SEG
S2 = S // 2
D2 = 2 * D
NT2 = NK * NK * S2

NC = 2
NS = 16
NW = NC * NS

R2 = B * S2
SLABS = 26
R2S = R2 // SLABS
PER_W = R2S // NW
CHUNK = 88
NCHUNK = PER_W // CHUNK


def _ln(emb, gam, bet):
    mean = jnp.mean(emb, axis=-1, keepdims=True)
    cent = emb - mean
    var = jnp.mean(cent * cent, axis=-1, keepdims=True)
    return cent * lax.rsqrt(var + 1e-5) * gam + bet


def _rows_kernel(xe_ref, xo_ref, se_ref, so_ref, rows_ref):
    ke = xe_ref[...] * N_SEG + se_ref[...]
    ko = xo_ref[...] * N_SEG + so_ref[...]
    s_iota = lax.broadcasted_iota(jnp.int32, (B, S2), 1)
    rows_ref[...] = (ke * NK + ko) * S2 + s_iota


def _tc_rows(xe, xo, se, so):
    return pl.pallas_call(
        _rows_kernel,
        out_shape=jax.ShapeDtypeStruct((B, S2), jnp.int32),
    )(xe, xo, se, so)


def _table_kernel(tok_ref, pos_e_ref, pos_o_ref, seg_ref, gam_ref, bet_ref,
                  tab_ref):
    k = pl.program_id(0)
    k1, k2 = k // NK, k % NK
    v1, g1 = k1 // N_SEG, k1 % N_SEG
    v2, g2 = k2 // N_SEG, k2 % N_SEG
    gam = gam_ref[0][None, :]
    bet = bet_ref[0][None, :]
    comb_e = tok_ref[pl.ds(v1, 1), :] + seg_ref[pl.ds(g1, 1), :]
    comb_o = tok_ref[pl.ds(v2, 1), :] + seg_ref[pl.ds(g2, 1), :]
    out_e = _ln(comb_e + pos_e_ref[...], gam, bet)
    out_o = _ln(comb_o + pos_o_ref[...], gam, bet)
    tab_ref[0] = jnp.concatenate([out_e, out_o], axis=-1)


def _tc_table(tok, pos_e, pos_o, sege, gam, bet):
    full = lambda shape: pl.BlockSpec(shape, lambda k: (0,) * len(shape))
    return pl.pallas_call(
        _table_kernel,
        grid=(NK * NK,),
        in_specs=[
            full((VOCAB, D)),
            full((S2, D)),
            full((S2, D)),
            full((N_SEG, D)),
            full((1, D)),
            full((1, D)),
        ],
        out_specs=pl.BlockSpec((1, S2, D2), lambda k: (k, 0, 0)),
        out_shape=jax.ShapeDtypeStruct((NK * NK, S2, D2), jnp.float32),
    )(tok, pos_e, pos_o, sege, gam, bet)


@functools.cache
def _make_sc_gather():
    mesh = plsc.VectorSubcoreMesh(core_axis_name="c", subcore_axis_name="s")

    @functools.partial(
        pl.kernel,
        mesh=mesh,
        out_type=jax.ShapeDtypeStruct((R2S, D2), jnp.float32),
        scratch_types=[
            pltpu.VMEM((NCHUNK, CHUNK), jnp.int32),
            pltpu.VMEM((CHUNK, D2), jnp.float32),
            pltpu.VMEM((CHUNK, D2), jnp.float32),
            pltpu.SemaphoreType.DMA,
            pltpu.SemaphoreType.DMA,
        ],
        compiler_params=pltpu.CompilerParams(use_tc_tiling_on_sc=False),
    )
    def _sc_gather(tab_hbm, rows_hbm, out_hbm, idx_v, buf0, buf1, sem0, sem1):
        wid = lax.axis_index("s") * NC + lax.axis_index("c")
        base = wid * PER_W
        pltpu.sync_copy(rows_hbm.at[wid], idx_v)

        bufs = (buf0, buf1)
        sems = (sem0, sem1)

        def body(j2, _):
            copies = []
            for b in range(2):
                j = j2 * 2 + b
                copies.append(pltpu.async_copy(
                    tab_hbm.at[idx_v.at[j]], bufs[b], sems[b]))
            for b in range(2):
                j = j2 * 2 + b
                copies[b].wait()
                pltpu.sync_copy(bufs[b],
                                out_hbm.at[pl.ds(base + j * CHUNK, CHUNK)])
            return 0

        lax.fori_loop(0, NCHUNK // 2, body, 0)

        if NCHUNK % 2:
            j = NCHUNK - 1
            pltpu.async_copy(tab_hbm.at[idx_v.at[j]], buf0, sem0).wait()
            pltpu.sync_copy(buf0, out_hbm.at[pl.ds(base + j * CHUNK, CHUNK)])

    return _sc_gather


def kernel(x, seg, tok_embed, pos_embed, seg_embed, gamma, beta):
    rows2 = _tc_rows(x[:, 0::2], x[:, 1::2], seg[:, 0::2], seg[:, 1::2])
    tab = _tc_table(tok_embed, pos_embed[0::2], pos_embed[1::2], seg_embed,
                    gamma.reshape(1, D), beta.reshape(1, D))
    tab2 = tab.reshape(NT2, D2)
    rows4 = rows2.reshape(SLABS, NW, NCHUNK, CHUNK)
    gather = _make_sc_gather()
    outs = [gather(tab2, rows4[i]) for i in range(SLABS)]
    out = jnp.concatenate(outs, axis=0)
    return out.reshape(B, S, D)


# speedup vs baseline: 1.3579x; 1.0508x over previous
import functools

import jax
import jax.numpy as jnp
from jax import lax
from jax.experimental import pallas as pl
from jax.experimental.pallas import tpu as pltpu
from jax.experimental.pallas import tpu_sc as plsc

VOCAB = 4
N_SEG = 2
B = 1024
S = 286
D = 284
NK = VOCAB * N_SEG
S2 = S // 2
D2 = 2 * D
NT2 = NK * NK * S2

NC = 2
NS = 16
NW = NC * NS

R2 = B * S2
SLABS = 11
R2S = R2 // SLABS
PER_W = R2S // NW
CHUNK = 104
NCHUNK = PER_W // CHUNK


def _ln(emb, gam, bet):
    mean = jnp.mean(emb, axis=-1, keepdims=True)
    cent = emb - mean
    var = jnp.mean(cent * cent, axis=-1, keepdims=True)
    return cent * lax.rsqrt(var + 1e-5) * gam + bet


def _rows_kernel(xe_ref, xo_ref, se_ref, so_ref, rows_ref):
    ke = xe_ref[...] * N_SEG + se_ref[...]
    ko = xo_ref[...] * N_SEG + so_ref[...]
    s_iota = lax.broadcasted_iota(jnp.int32, (B, S2), 1)
    rows_ref[...] = (ke * NK + ko) * S2 + s_iota


def _tc_rows(xe, xo, se, so):
    return pl.pallas_call(
        _rows_kernel,
        out_shape=jax.ShapeDtypeStruct((B, S2), jnp.int32),
    )(xe, xo, se, so)


def _table_kernel(tok_ref, pos_e_ref, pos_o_ref, seg_ref, gam_ref, bet_ref,
                  tab_ref):
    k = pl.program_id(0)
    k1, k2 = k // NK, k % NK
    v1, g1 = k1 // N_SEG, k1 % N_SEG
    v2, g2 = k2 // N_SEG, k2 % N_SEG
    gam = gam_ref[0][None, :]
    bet = bet_ref[0][None, :]
    comb_e = tok_ref[pl.ds(v1, 1), :] + seg_ref[pl.ds(g1, 1), :]
    comb_o = tok_ref[pl.ds(v2, 1), :] + seg_ref[pl.ds(g2, 1), :]
    out_e = _ln(comb_e + pos_e_ref[...], gam, bet)
    out_o = _ln(comb_o + pos_o_ref[...], gam, bet)
    tab_ref[0] = jnp.concatenate([out_e, out_o], axis=-1)


def _tc_table(tok, pos_e, pos_o, sege, gam, bet):
    full = lambda shape: pl.BlockSpec(shape, lambda k: (0,) * len(shape))
    return pl.pallas_call(
        _table_kernel,
        grid=(NK * NK,),
        in_specs=[
            full((VOCAB, D)),
            full((S2, D)),
            full((S2, D)),
            full((N_SEG, D)),
            full((1, D)),
            full((1, D)),
        ],
        out_specs=pl.BlockSpec((1, S2, D2), lambda k: (k, 0, 0)),
        out_shape=jax.ShapeDtypeStruct((NK * NK, S2, D2), jnp.float32),
    )(tok, pos_e, pos_o, sege, gam, bet)


@functools.cache
def _make_sc_gather():
    mesh = plsc.VectorSubcoreMesh(core_axis_name="c", subcore_axis_name="s")

    @functools.partial(
        pl.kernel,
        mesh=mesh,
        out_type=jax.ShapeDtypeStruct((R2S, D2), jnp.float32),
        scratch_types=[
            pltpu.VMEM((NCHUNK, CHUNK), jnp.int32),
            pltpu.VMEM((CHUNK, D2), jnp.float32),
            pltpu.VMEM((CHUNK, D2), jnp.float32),
            pltpu.SemaphoreType.DMA,
            pltpu.SemaphoreType.DMA,
        ],
        compiler_params=pltpu.CompilerParams(use_tc_tiling_on_sc=False),
    )
    def _sc_gather(tab_hbm, rows_hbm, out_hbm, idx_v, buf0, buf1, sem0, sem1):
        wid = lax.axis_index("s") * NC + lax.axis_index("c")
        base = wid * PER_W
        pltpu.sync_copy(rows_hbm.at[wid], idx_v)

        bufs = (buf0, buf1)
        sems = (sem0, sem1)

        def body(j2, _):
            copies = []
            for b in range(2):
                j = j2 * 2 + b
                copies.append(pltpu.async_copy(
                    tab_hbm.at[idx_v.at[j]], bufs[b], sems[b]))
            for b in range(2):
                j = j2 * 2 + b
                copies[b].wait()
                pltpu.sync_copy(bufs[b],
                                out_hbm.at[pl.ds(base + j * CHUNK, CHUNK)])
            return 0

        lax.fori_loop(0, NCHUNK // 2, body, 0)

        if NCHUNK % 2:
            j = NCHUNK - 1
            pltpu.async_copy(tab_hbm.at[idx_v.at[j]], buf0, sem0).wait()
            pltpu.sync_copy(buf0, out_hbm.at[pl.ds(base + j * CHUNK, CHUNK)])

    return _sc_gather


def kernel(x, seg, tok_embed, pos_embed, seg_embed, gamma, beta):
    rows2 = _tc_rows(x[:, 0::2], x[:, 1::2], seg[:, 0::2], seg[:, 1::2])
    tab = _tc_table(tok_embed, pos_embed[0::2], pos_embed[1::2], seg_embed,
                    gamma.reshape(1, D), beta.reshape(1, D))
    tab2 = tab.reshape(NT2, D2)
    rows4 = rows2.reshape(SLABS, NW, NCHUNK, CHUNK)
    gather = _make_sc_gather()
    outs = [gather(tab2, rows4[i]) for i in range(SLABS)]
    out = jnp.concatenate(outs, axis=0)
    return out.reshape(B, S, D)
